# baseline (speedup 1.0000x reference)
import functools

import jax
import jax.numpy as jnp
from jax import lax
from jax.experimental import pallas as pl
from jax.experimental.pallas import tpu as pltpu
from jax.experimental.pallas import tpu_sc as plsc

NUM_CORES = 2
NUM_SUBCORES = 16
NUM_WORKERS = NUM_CORES * NUM_SUBCORES
BLK = 128
LANES = 16
STEPS_C0 = 63
STEPS_C1 = 95
CHUNK = 48
NCH = 2
SMAX = CHUNK * NCH
ROWS_PER_TILE = 640
NPAD = NUM_SUBCORES * ROWS_PER_TILE


def _sc_aggregate(x, rowp, colp, valp):
    n, d = x.shape
    nvec = d // LANES
    nz = ROWS_PER_TILE // BLK
    mesh = plsc.VectorSubcoreMesh(core_axis_name="c", subcore_axis_name="s")

    @functools.partial(
        pl.kernel,
        out_type=jax.ShapeDtypeStruct((NUM_CORES, NPAD, d), jnp.float32),
        mesh=mesh,
        scratch_types=[
            pltpu.VMEM((CHUNK, BLK), jnp.int32),
            pltpu.VMEM((CHUNK, BLK), jnp.int32),
            pltpu.VMEM((CHUNK, BLK), jnp.float32),
            pltpu.VMEM((BLK, d), jnp.float32),
            pltpu.VMEM_SHARED((NPAD, d), jnp.float32),
            pltpu.SemaphoreType.DMA,
        ],
    )
    def body(x_hbm, rowp_hbm, colp_hbm, valp_hbm, out_hbm,
             row_v, col_v, val_v, gath, acc, sem):
        c = lax.axis_index("c")
        s = lax.axis_index("s")
        wid = s * NUM_CORES + c

        def zero_body(i, carry):
            for k in range(nvec):
                gath[i, pl.ds(k * LANES, LANES)] = jnp.zeros((LANES,), jnp.float32)
            return carry

        lax.fori_loop(0, BLK, zero_body, 0)
        base = s * ROWS_PER_TILE
        for k in range(nz):
            pltpu.sync_copy(gath, acc.at[pl.ds(base + k * BLK, BLK)])
        plsc.subcore_barrier()

        def step_body(t, carry):
            pltpu.async_copy(x_hbm.at[col_v.at[t]], gath, sem).wait()

            def scale_group(g, c2):
                vblock = val_v[t, pl.ds(g * LANES, LANES)]
                ebase = g * LANES
                for j in range(LANES):
                    v = vblock[j]
                    for k in range(nvec):
                        sl = pl.ds(k * LANES, LANES)
                        gath[ebase + j, sl] = gath[ebase + j, sl] * v
                return c2

            lax.fori_loop(0, BLK // LANES, scale_group, 0)
            pltpu.sync_copy(gath, acc.at[row_v.at[t]], add=True)
            return carry

        nsteps = STEPS_C0 + (STEPS_C1 - STEPS_C0) * c
        for h in range(NCH):
            pltpu.sync_copy(rowp_hbm.at[wid, h], row_v)
            pltpu.sync_copy(colp_hbm.at[wid, h], col_v)
            pltpu.sync_copy(valp_hbm.at[wid, h], val_v)
            bound = jnp.clip(nsteps - h * CHUNK, 0, CHUNK)
            lax.fori_loop(0, bound, step_body, 0)
        plsc.subcore_barrier()
        sl = pl.ds(base, ROWS_PER_TILE)
        pltpu.sync_copy(acc.at[sl], out_hbm.at[c, sl])

    return body(x, rowp, colp, valp)


def _tc_linear(x, partials, w, b2):
    n, d = x.shape
    bn = 1000

    def body(x_ref, p_ref, w_ref, b_ref, o_ref):
        xb = x_ref[...]
        nb = p_ref[0] + p_ref[1]
        w1 = w_ref[:, :d]
        w2 = w_ref[:, d:]
        acc = lax.dot_general(xb, w1, (((1,), (1,)), ((), ())),
                              preferred_element_type=jnp.float32)
        acc = acc + lax.dot_general(nb, w2, (((1,), (1,)), ((), ())),
                                    preferred_element_type=jnp.float32)
        o_ref[...] = acc + b_ref[...]

    return pl.pallas_call(
        body,
        grid=(n // bn,),
        in_specs=[
            pl.BlockSpec((bn, d), lambda i: (i, 0)),
            pl.BlockSpec((NUM_CORES, bn, d), lambda i: (0, i, 0)),
            pl.BlockSpec((d, 2 * d), lambda i: (0, 0)),
            pl.BlockSpec((1, d), lambda i: (0, 0)),
        ],
        out_specs=pl.BlockSpec((bn, d), lambda i: (i, 0)),
        out_shape=jax.ShapeDtypeStruct((n, d), jnp.float32),
    )(x, partials, w, b2)


def _partition(a, counts, fill):
    segs = []
    off = 0
    cap = SMAX * BLK
    for cnt in counts:
        seg = a[off:off + cnt]
        if cnt < cap:
            seg = jnp.concatenate([seg, jnp.full((cap - cnt,), fill, a.dtype)])
        segs.append(seg)
        off += cnt
    return jnp.stack(segs).reshape(NUM_WORKERS, NCH, CHUNK, BLK)


def kernel(x, adj_indices, adj_values, W, b):
    n, d = x.shape
    e = adj_values.shape[0]
    row = adj_indices[0]
    col = adj_indices[1]

    counts = [(STEPS_C0 if w % 2 == 0 else STEPS_C1) * BLK
              for w in range(NUM_WORKERS)]
    total = sum(counts)
    pad = total - e
    row = jnp.concatenate([row, jnp.zeros((pad,), row.dtype)])
    col = jnp.concatenate([col, jnp.zeros((pad,), col.dtype)])
    val = jnp.concatenate([adj_values, jnp.zeros((pad,), adj_values.dtype)])

    rowp = _partition(row, counts, 0)
    colp = _partition(col, counts, 0)
    valp = _partition(val, counts, 0)

    partials = _sc_aggregate(x, rowp, colp, valp)
    return _tc_linear(x, partials, W, b.reshape(1, d))


# speedup vs baseline: 1.1731x; 1.1731x over previous
import functools

import jax
import jax.numpy as jnp
from jax import lax
from jax.experimental import pallas as pl
from jax.experimental.pallas import tpu as pltpu
from jax.experimental.pallas import tpu_sc as plsc

NUM_CORES = 2
NUM_SUBCORES = 16
NUM_WORKERS = NUM_CORES * NUM_SUBCORES
BLK = 128
LANES = 16
STEPS_C0 = 95
STEPS_C1 = 63
CHUNK = 48
NCH = 2
SMAX = CHUNK * NCH
ROWS_PER_TILE = 640
NPAD = NUM_SUBCORES * ROWS_PER_TILE


def _sc_aggregate(x, rowp, colp, valp):
    n, d = x.shape
    nvec = d // LANES
    nz = ROWS_PER_TILE // BLK
    mesh = plsc.VectorSubcoreMesh(core_axis_name="c", subcore_axis_name="s")

    @functools.partial(
        pl.kernel,
        out_type=jax.ShapeDtypeStruct((NUM_CORES, NPAD, d), jnp.float32),
        mesh=mesh,
        scratch_types=[
            pltpu.VMEM((CHUNK, BLK), jnp.int32),
            pltpu.VMEM((CHUNK, BLK), jnp.int32),
            pltpu.VMEM((CHUNK, BLK), jnp.float32),
            pltpu.VMEM((BLK, d), jnp.float32),
            pltpu.VMEM_SHARED((NPAD, d), jnp.float32),
            pltpu.SemaphoreType.DMA,
        ],
    )
    def body(x_hbm, rowp_hbm, colp_hbm, valp_hbm, out_hbm,
             row_v, col_v, val_v, gath, acc, sem):
        c = lax.axis_index("c")
        s = lax.axis_index("s")
        wid = s * NUM_CORES + c

        def zero_body(i, carry):
            for k in range(nvec):
                gath[i, pl.ds(k * LANES, LANES)] = jnp.zeros((LANES,), jnp.float32)
            return carry

        lax.fori_loop(0, BLK, zero_body, 0)
        base = s * ROWS_PER_TILE
        for k in range(nz):
            pltpu.sync_copy(gath, acc.at[pl.ds(base + k * BLK, BLK)])
        plsc.subcore_barrier()

        def step_body(t, carry):
            pltpu.async_copy(x_hbm.at[col_v.at[t]], gath, sem).wait()

            def scale_group(g, c2):
                vblock = val_v[t, pl.ds(g * LANES, LANES)]
                ebase = g * LANES
                for j in range(LANES):
                    v = vblock[j]
                    for k in range(nvec):
                        sl = pl.ds(k * LANES, LANES)
                        gath[ebase + j, sl] = gath[ebase + j, sl] * v
                return c2

            lax.fori_loop(0, BLK // LANES, scale_group, 0)
            pltpu.sync_copy(gath, acc.at[row_v.at[t]], add=True)
            return carry

        nsteps = STEPS_C0 + (STEPS_C1 - STEPS_C0) * c
        for h in range(NCH):
            pltpu.sync_copy(rowp_hbm.at[wid, h], row_v)
            pltpu.sync_copy(colp_hbm.at[wid, h], col_v)
            pltpu.sync_copy(valp_hbm.at[wid, h], val_v)
            bound = jnp.clip(nsteps - h * CHUNK, 0, CHUNK)
            lax.fori_loop(0, bound, step_body, 0)
        plsc.subcore_barrier()
        sl = pl.ds(base, ROWS_PER_TILE)
        pltpu.sync_copy(acc.at[sl], out_hbm.at[c, sl])

    return body(x, rowp, colp, valp)


def _tc_linear(x, partials, w, b2):
    n, d = x.shape
    bn = 1000

    def body(x_ref, p_ref, w_ref, b_ref, o_ref):
        xb = x_ref[...]
        nb = p_ref[0] + p_ref[1]
        w1 = w_ref[:, :d]
        w2 = w_ref[:, d:]
        acc = lax.dot_general(xb, w1, (((1,), (1,)), ((), ())),
                              preferred_element_type=jnp.float32)
        acc = acc + lax.dot_general(nb, w2, (((1,), (1,)), ((), ())),
                                    preferred_element_type=jnp.float32)
        o_ref[...] = acc + b_ref[...]

    return pl.pallas_call(
        body,
        grid=(n // bn,),
        in_specs=[
            pl.BlockSpec((bn, d), lambda i: (i, 0)),
            pl.BlockSpec((NUM_CORES, bn, d), lambda i: (0, i, 0)),
            pl.BlockSpec((d, 2 * d), lambda i: (0, 0)),
            pl.BlockSpec((1, d), lambda i: (0, 0)),
        ],
        out_specs=pl.BlockSpec((bn, d), lambda i: (i, 0)),
        out_shape=jax.ShapeDtypeStruct((n, d), jnp.float32),
    )(x, partials, w, b2)


def _partition(a, counts, fill):
    segs = []
    off = 0
    cap = SMAX * BLK
    for cnt in counts:
        seg = a[off:off + cnt]
        if cnt < cap:
            seg = jnp.concatenate([seg, jnp.full((cap - cnt,), fill, a.dtype)])
        segs.append(seg)
        off += cnt
    return jnp.stack(segs).reshape(NUM_WORKERS, NCH, CHUNK, BLK)


def kernel(x, adj_indices, adj_values, W, b):
    n, d = x.shape
    e = adj_values.shape[0]
    row = adj_indices[0]
    col = adj_indices[1]

    counts = [(STEPS_C0 if w % 2 == 0 else STEPS_C1) * BLK
              for w in range(NUM_WORKERS)]
    total = sum(counts)
    pad = total - e
    row = jnp.concatenate([row, jnp.zeros((pad,), row.dtype)])
    col = jnp.concatenate([col, jnp.zeros((pad,), col.dtype)])
    val = jnp.concatenate([adj_values, jnp.zeros((pad,), adj_values.dtype)])

    rowp = _partition(row, counts, 0)
    colp = _partition(col, counts, 0)
    valp = _partition(val, counts, 0)

    partials = _sc_aggregate(x, rowp, colp, valp)
    return _tc_linear(x, partials, W, b.reshape(1, d))
